# baseline (speedup 1.0000x reference)
import functools

import jax
import jax.numpy as jnp
from jax.experimental import pallas as pl
from jax.experimental.pallas import tpu as pltpu

_TOP_K = (1, 5)
_THR = 0.0


def _gather_body(cg, y_ref, x_ref, t_ref):
    i = pl.program_id(0)
    off = y_ref[i] - (y_ref[i] // cg) * cg
    lane = jax.lax.broadcasted_iota(jnp.int32, (1, cg), 1)
    picked = jnp.sum(jnp.where(lane == off, x_ref[0, 0, :][None, :], 0.0),
                     axis=1, keepdims=True)
    t_ref[...] = picked[None]


def _gather_targets(y_hat, y):
    B, V = y_hat.shape
    cg = min(512, ((V + 127) // 128) * 128)
    x3 = y_hat.reshape(B, 1, V)
    grid_spec = pltpu.PrefetchScalarGridSpec(
        num_scalar_prefetch=1,
        grid=(B,),
        in_specs=[
            pl.BlockSpec((1, 1, cg), lambda i, yp: (i, 0, yp[i] // cg)),
        ],
        out_specs=pl.BlockSpec((1, 1, 1), lambda i, yp: (i, 0, 0)),
    )
    out = pl.pallas_call(
        functools.partial(_gather_body, cg),
        grid_spec=grid_spec,
        out_shape=jax.ShapeDtypeStruct((B, 1, 1), jnp.float32),
    )(y, x3)
    return out.reshape(B, 1)


def _rank_body(nc, nr, c, v_total, num, y_ref, t_ref, x_ref, acc_ref, rank_ref):
    i = pl.program_id(0)
    j = pl.program_id(1)

    @pl.when(j == 0)
    def _init():
        rank_ref[...] = jnp.zeros_like(rank_ref)

    v = x_ref[...]
    t = t_ref[...]
    yy = y_ref[...]
    col = j * c + jax.lax.broadcasted_iota(jnp.int32, v.shape, 1)
    beats = (v > t) | ((v == t) & (col < yy))
    valid = col < v_total
    cnt = jnp.sum(jnp.where(valid & beats, 1, 0), axis=1, keepdims=True)
    rank_ref[...] += cnt

    @pl.when(j == nc - 1)
    def _fin():
        @pl.when(i == 0)
        def _zero():
            acc_ref[...] = jnp.zeros_like(acc_ref)

        rank = rank_ref[...]
        tpos = t_ref[...] > _THR
        lane2 = jax.lax.broadcasted_iota(jnp.int32, (1, 2), 1)
        c1 = jnp.sum(jnp.where((rank < _TOP_K[0]) & tpos, 1.0, 0.0))
        c5 = jnp.sum(jnp.where((rank < _TOP_K[1]) & tpos, 1.0, 0.0))
        acc_ref[...] += jnp.where(lane2 == 0, c1, c5)

        @pl.when(i == nr - 1)
        def _scale():
            acc_ref[...] = acc_ref[...] * (100.0 / num)


def _rank_accuracy(y_hat, y, t):
    B, V = y_hat.shape
    R = min(512, B)
    nr = B // R
    C = min(4096, ((V + 127) // 128) * 128)
    nc = (V + C - 1) // C
    acc = pl.pallas_call(
        functools.partial(_rank_body, nc, nr, C, V, B),
        grid=(nr, nc),
        in_specs=[
            pl.BlockSpec((R, 1), lambda i, j: (i, 0)),
            pl.BlockSpec((R, 1), lambda i, j: (i, 0)),
            pl.BlockSpec((R, C), lambda i, j: (i, j)),
        ],
        out_specs=pl.BlockSpec((1, 2), lambda i, j: (0, 0)),
        out_shape=jax.ShapeDtypeStruct((1, 2), jnp.float32),
        scratch_shapes=[pltpu.VMEM((R, 1), jnp.int32)],
    )(y.reshape(B, 1), t, y_hat)
    return acc


def kernel(y_hat, y):
    B, V = y_hat.shape
    y = y.astype(jnp.int32)
    t = _gather_targets(y_hat, y)
    acc = _rank_accuracy(y_hat, y, t)
    return (acc[0, 0:1], acc[0, 1:2])


# speedup vs baseline: 1.6900x; 1.6900x over previous
import functools

import jax
import jax.numpy as jnp
from jax import lax
from jax.experimental import pallas as pl
from jax.experimental.pallas import tpu as pltpu
from jax.experimental.pallas import tpu_sc as plsc

_TOP_K = (1, 5)
_THR = 0.0


def _gather_targets(y_hat, y):
    B, V = y_hat.shape
    info = plsc.get_sparse_core_info()
    ncores, nsub, L = info.num_cores, info.num_subcores, info.num_lanes
    nw = ncores * nsub
    bw = B // nw
    flat = y_hat.reshape(B * V)
    mesh = plsc.VectorSubcoreMesh(core_axis_name="c", subcore_axis_name="s")

    @functools.partial(
        pl.kernel, mesh=mesh,
        out_type=jax.ShapeDtypeStruct((B,), jnp.float32),
        scratch_types=[
            pltpu.VMEM((bw,), jnp.int32),
            pltpu.VMEM((bw,), jnp.float32),
            pltpu.SemaphoreType.DMA,
        ],
    )
    def gat(flat_hbm, y_hbm, out_hbm, idx_v, vals_v, sem):
        wid = lax.axis_index("s") * ncores + lax.axis_index("c")
        base = wid * bw
        pltpu.sync_copy(y_hbm.at[pl.ds(base, bw)], idx_v)
        for k in range(bw // L):
            row0 = base + k * L
            off = (row0 + lax.iota(jnp.int32, L)) * V
            sl = pl.ds(k * L, L)
            idx_v[sl] = idx_v[sl] + off
        pltpu.async_copy(flat_hbm.at[idx_v], vals_v, sem).wait()
        pltpu.sync_copy(vals_v, out_hbm.at[pl.ds(base, bw)])

    return gat(flat, y).reshape(B, 1)


def _rank_body(nc, nr, c, v_total, num, y_ref, t_ref, x_ref, acc_ref, rank_ref):
    i = pl.program_id(0)
    j = pl.program_id(1)

    @pl.when(j == 0)
    def _init():
        rank_ref[...] = jnp.zeros_like(rank_ref)

    v = x_ref[...]
    t = t_ref[...]
    yy = y_ref[...]
    col = j * c + jax.lax.broadcasted_iota(jnp.int32, v.shape, 1)
    beats = (v > t) | ((v == t) & (col < yy))
    valid = col < v_total
    cnt = jnp.sum(jnp.where(valid & beats, 1, 0), axis=1, keepdims=True)
    rank_ref[...] += cnt

    @pl.when(j == nc - 1)
    def _fin():
        @pl.when(i == 0)
        def _zero():
            acc_ref[...] = jnp.zeros_like(acc_ref)

        rank = rank_ref[...]
        tpos = t_ref[...] > _THR
        lane2 = jax.lax.broadcasted_iota(jnp.int32, (1, 2), 1)
        c1 = jnp.sum(jnp.where((rank < _TOP_K[0]) & tpos, 1.0, 0.0))
        c5 = jnp.sum(jnp.where((rank < _TOP_K[1]) & tpos, 1.0, 0.0))
        acc_ref[...] += jnp.where(lane2 == 0, c1, c5)

        @pl.when(i == nr - 1)
        def _scale():
            acc_ref[...] = acc_ref[...] * (100.0 / num)


def _rank_accuracy(y_hat, y, t):
    B, V = y_hat.shape
    R = min(512, B)
    nr = B // R
    C = min(4096, ((V + 127) // 128) * 128)
    nc = (V + C - 1) // C
    acc = pl.pallas_call(
        functools.partial(_rank_body, nc, nr, C, V, B),
        grid=(nr, nc),
        in_specs=[
            pl.BlockSpec((R, 1), lambda i, j: (i, 0)),
            pl.BlockSpec((R, 1), lambda i, j: (i, 0)),
            pl.BlockSpec((R, C), lambda i, j: (i, j)),
        ],
        out_specs=pl.BlockSpec((1, 2), lambda i, j: (0, 0)),
        out_shape=jax.ShapeDtypeStruct((1, 2), jnp.float32),
        scratch_shapes=[pltpu.VMEM((R, 1), jnp.int32)],
    )(y.reshape(B, 1), t, y_hat)
    return acc


def kernel(y_hat, y):
    B, V = y_hat.shape
    y = y.astype(jnp.int32)
    t = _gather_targets(y_hat, y)
    acc = _rank_accuracy(y_hat, y, t)
    return (acc[0, 0:1], acc[0, 1:2])


# speedup vs baseline: 1.6905x; 1.0003x over previous
import functools

import jax
import jax.numpy as jnp
from jax import lax
from jax.experimental import pallas as pl
from jax.experimental.pallas import tpu as pltpu
from jax.experimental.pallas import tpu_sc as plsc

_TOP_K = (1, 5)
_THR = 0.0


def _gather_targets(y_hat, y):
    B, V = y_hat.shape
    info = plsc.get_sparse_core_info()
    ncores, nsub, L = info.num_cores, info.num_subcores, info.num_lanes
    nw = ncores * nsub
    bw = B // nw
    flat = y_hat.reshape(B * V)
    mesh = plsc.VectorSubcoreMesh(core_axis_name="c", subcore_axis_name="s")

    @functools.partial(
        pl.kernel, mesh=mesh,
        out_type=jax.ShapeDtypeStruct((B,), jnp.float32),
        scratch_types=[
            pltpu.VMEM((bw,), jnp.int32),
            pltpu.VMEM((bw,), jnp.float32),
            pltpu.SemaphoreType.DMA,
        ],
    )
    def gat(flat_hbm, y_hbm, out_hbm, idx_v, vals_v, sem):
        wid = lax.axis_index("s") * ncores + lax.axis_index("c")
        base = wid * bw
        pltpu.sync_copy(y_hbm.at[pl.ds(base, bw)], idx_v)
        for k in range(bw // L):
            row0 = base + k * L
            off = (row0 + lax.iota(jnp.int32, L)) * V
            sl = pl.ds(k * L, L)
            idx_v[sl] = idx_v[sl] + off
        pltpu.async_copy(flat_hbm.at[idx_v], vals_v, sem).wait()
        pltpu.sync_copy(vals_v, out_hbm.at[pl.ds(base, bw)])

    return gat(flat, y).reshape(B, 1)


def _rank_body(nc, nr, c, v_total, num, y_ref, t_ref, x_ref, acc_ref, rank_ref):
    i = pl.program_id(0)
    j = pl.program_id(1)

    @pl.when(j == 0)
    def _init():
        rank_ref[...] = jnp.zeros_like(rank_ref)

    v = x_ref[...]
    t = t_ref[...]
    yy = y_ref[...]
    col = j * c + jax.lax.broadcasted_iota(jnp.int32, v.shape, 1)
    beats = (v > t) | ((v == t) & (col < yy))
    valid = col < v_total
    cnt = jnp.sum(jnp.where(valid & beats, 1, 0), axis=1, keepdims=True)
    rank_ref[...] += cnt

    @pl.when(j == nc - 1)
    def _fin():
        @pl.when(i == 0)
        def _zero():
            acc_ref[...] = jnp.zeros_like(acc_ref)

        rank = rank_ref[...]
        tpos = t_ref[...] > _THR
        lane2 = jax.lax.broadcasted_iota(jnp.int32, (1, 2), 1)
        c1 = jnp.sum(jnp.where((rank < _TOP_K[0]) & tpos, 1.0, 0.0))
        c5 = jnp.sum(jnp.where((rank < _TOP_K[1]) & tpos, 1.0, 0.0))
        acc_ref[...] += jnp.where(lane2 == 0, c1, c5)

        @pl.when(i == nr - 1)
        def _scale():
            acc_ref[...] = acc_ref[...] * (100.0 / num)


def _rank_accuracy(y_hat, y, t):
    B, V = y_hat.shape
    R = min(512, B)
    nr = B // R
    C = min(8192, ((V + 127) // 128) * 128)
    nc = (V + C - 1) // C
    acc = pl.pallas_call(
        functools.partial(_rank_body, nc, nr, C, V, B),
        grid=(nr, nc),
        in_specs=[
            pl.BlockSpec((R, 1), lambda i, j: (i, 0)),
            pl.BlockSpec((R, 1), lambda i, j: (i, 0)),
            pl.BlockSpec((R, C), lambda i, j: (i, j)),
        ],
        out_specs=pl.BlockSpec((1, 2), lambda i, j: (0, 0)),
        out_shape=jax.ShapeDtypeStruct((1, 2), jnp.float32),
        scratch_shapes=[pltpu.VMEM((R, 1), jnp.int32)],
    )(y.reshape(B, 1), t, y_hat)
    return acc


def kernel(y_hat, y):
    B, V = y_hat.shape
    y = y.astype(jnp.int32)
    t = _gather_targets(y_hat, y)
    acc = _rank_accuracy(y_hat, y, t)
    return (acc[0, 0:1], acc[0, 1:2])
